# baseline (speedup 1.0000x reference)
import jax
import jax.numpy as jnp
from jax.experimental import pallas as pl

_BLK = 512


def _max_body(a_ref, b_ref, out_ref):
    out_ref[...] = jnp.maximum(
        jnp.max(a_ref[...], axis=1), jnp.max(b_ref[...], axis=1)
    )


def kernel(mailbox):
    n, m, d = mailbox.shape
    h = m // 2
    grid = (-(-n // _BLK),)
    return pl.pallas_call(
        _max_body,
        grid=grid,
        in_specs=[
            pl.BlockSpec((_BLK, h, d), lambda i: (i, 0, 0)),
            pl.BlockSpec((_BLK, h, d), lambda i: (i, 1, 0)),
        ],
        out_specs=pl.BlockSpec((_BLK, d), lambda i: (i, 0)),
        out_shape=jax.ShapeDtypeStruct((n, d), mailbox.dtype),
    )(mailbox, mailbox)


# speedup vs baseline: 1.0971x; 1.0971x over previous
import jax
import jax.numpy as jnp
from jax.experimental import pallas as pl

_BLK = 512


def _max_body(mail_ref, out_ref):
    out_ref[...] = jnp.max(mail_ref[...], axis=1)


def kernel(mailbox):
    n, m, d = mailbox.shape
    grid = (-(-n // _BLK),)
    return pl.pallas_call(
        _max_body,
        grid=grid,
        in_specs=[pl.BlockSpec((_BLK, m, d), lambda i: (i, 0, 0))],
        out_specs=pl.BlockSpec((_BLK, d), lambda i: (i, 0)),
        out_shape=jax.ShapeDtypeStruct((n, d), mailbox.dtype),
    )(mailbox)
